# baseline (speedup 1.0000x reference)
import jax
import jax.numpy as jnp
from jax import lax
from jax.experimental import pallas as pl
from jax.experimental.pallas import tpu as pltpu
from jax.experimental.pallas import tpu_sc as plsc

_NW = 32
_CHUNK = 128


def kernel(weeks, seasonal_bias):
    n = weeks.shape[0]
    v, d0, d1 = seasonal_bias.shape
    per_w = n // _NW
    chunks = per_w // _CHUNK
    idx = weeks.astype(jnp.int32)

    mesh = plsc.VectorSubcoreMesh(core_axis_name="c", subcore_axis_name="s")

    @pl.kernel(
        out_type=jax.ShapeDtypeStruct((n, d0, d1), seasonal_bias.dtype),
        mesh=mesh,
        scratch_types=[
            pltpu.VMEM_SHARED((v, d0, d1), jnp.float32),
            pltpu.VMEM((per_w,), jnp.int32),
            pltpu.VMEM((per_w, d0, d1), jnp.float32),
            pltpu.SemaphoreType.DMA((chunks,)),
            pltpu.SemaphoreType.DMA((chunks,)),
            pltpu.SemaphoreType.DMA,
        ],
        compiler_params=pltpu.CompilerParams(use_tc_tiling_on_sc=False),
    )
    def gather_kernel(
        table_hbm, idx_hbm, out_hbm, table_v, idx_v, rows_v, gsem, osem, tsem
    ):
        wid = lax.axis_index("s") * 2 + lax.axis_index("c")
        base = wid * per_w
        tcopy = pltpu.make_async_copy(table_hbm, table_v, tsem)
        tcopy.start()
        pltpu.sync_copy(idx_hbm.at[pl.ds(base, per_w)], idx_v)
        tcopy.wait()

        gathers = []
        for k in range(chunks):
            g = pltpu.make_async_copy(
                table_v.at[idx_v.at[pl.ds(k * _CHUNK, _CHUNK)]],
                rows_v.at[pl.ds(k * _CHUNK, _CHUNK)],
                gsem.at[k],
            )
            g.start()
            gathers.append(g)
        outs = []
        for k in range(chunks):
            gathers[k].wait()
            o = pltpu.make_async_copy(
                rows_v.at[pl.ds(k * _CHUNK, _CHUNK)],
                out_hbm.at[pl.ds(base + k * _CHUNK, _CHUNK)],
                osem.at[k],
            )
            o.start()
            outs.append(o)
        for o in outs:
            o.wait()

    return gather_kernel(seasonal_bias, idx)


# speedup vs baseline: 1.0916x; 1.0916x over previous
import jax
import jax.numpy as jnp
from jax import lax
from jax.experimental import pallas as pl
from jax.experimental.pallas import tpu as pltpu
from jax.experimental.pallas import tpu_sc as plsc

_NW = 32


def kernel(weeks, seasonal_bias):
    n = weeks.shape[0]
    v, d0, d1 = seasonal_bias.shape
    per_w = n // _NW
    idx = weeks.astype(jnp.int32)

    mesh = plsc.VectorSubcoreMesh(core_axis_name="c", subcore_axis_name="s")

    @pl.kernel(
        out_type=jax.ShapeDtypeStruct((n, d0, d1), seasonal_bias.dtype),
        mesh=mesh,
        scratch_types=[
            pltpu.VMEM((v, d0, d1), jnp.float32),
            pltpu.VMEM_SHARED((n,), jnp.int32),
            pltpu.SMEM((per_w,), jnp.int32),
            pltpu.SemaphoreType.DMA,
            pltpu.SemaphoreType.DMA,
        ],
    )
    def gather_kernel(
        table_hbm, idx_hbm, out_hbm, table_v, idx_v, idx_s, osem, tsem
    ):
        wid = lax.axis_index("s") * 2 + lax.axis_index("c")
        base = wid * per_w
        tcopy = pltpu.make_async_copy(table_hbm, table_v, tsem)
        tcopy.start()
        pltpu.sync_copy(idx_hbm, idx_v)
        pltpu.sync_copy(idx_v.at[pl.ds(base, per_w)], idx_s)
        tcopy.wait()

        @pl.loop(0, per_w)
        def _(j):
            w = idx_s[j]
            pltpu.make_async_copy(
                table_v.at[w], out_hbm.at[base + j], osem
            ).start()

        @pl.loop(0, per_w)
        def _(j):
            pltpu.make_async_copy(
                table_v.at[0], out_hbm.at[base], osem
            ).wait()

    return gather_kernel(seasonal_bias, idx)


# speedup vs baseline: 1.8482x; 1.6932x over previous
import jax
import jax.numpy as jnp
from jax import lax
from jax.experimental import pallas as pl
from jax.experimental.pallas import tpu as pltpu
from jax.experimental.pallas import tpu_sc as plsc

_NW = 32
_L = 16


def kernel(weeks, seasonal_bias):
    n = weeks.shape[0]
    v, d0, d1 = seasonal_bias.shape
    planes = d0 * d1
    per_w = n // _NW
    idx = weeks.astype(jnp.int32)
    table_t = seasonal_bias.reshape(v, planes).T.reshape(-1)

    mesh = plsc.VectorSubcoreMesh(core_axis_name="c", subcore_axis_name="s")

    @pl.kernel(
        out_type=jax.ShapeDtypeStruct((d0, d1, n), seasonal_bias.dtype),
        mesh=mesh,
        scratch_types=[
            pltpu.VMEM((planes * v,), jnp.float32),
            pltpu.VMEM((per_w,), jnp.int32),
            pltpu.VMEM((d0, d1, per_w), jnp.float32),
            pltpu.SemaphoreType.DMA((d0,)),
            pltpu.SemaphoreType.DMA,
        ],
        compiler_params=pltpu.CompilerParams(needs_layout_passes=False),
    )
    def gather_kernel(
        table_hbm, idx_hbm, out_hbm, table_v, idx_v, pane_v, osem, tsem
    ):
        wid = lax.axis_index("s") * 2 + lax.axis_index("c")
        base = wid * per_w
        tcopy = pltpu.make_async_copy(table_hbm, table_v, tsem)
        tcopy.start()
        pltpu.sync_copy(idx_hbm.at[pl.ds(base, per_w)], idx_v)
        tcopy.wait()

        @pl.loop(0, per_w // _L)
        def _(c):
            w = idx_v[pl.ds(c * _L, _L)]
            for p in range(planes):
                vals = plsc.load_gather(table_v, [w + p * v])
                pane_v[p // d1, p % d1, pl.ds(c * _L, _L)] = vals

        outs = []
        for d in range(d0):
            o = pltpu.make_async_copy(
                pane_v.at[d],
                out_hbm.at[d, :, pl.ds(base, per_w)],
                osem.at[d],
            )
            o.start()
            outs.append(o)
        for o in outs:
            o.wait()

    out_t = gather_kernel(table_t, idx)
    return jnp.transpose(out_t, (2, 0, 1))


# speedup vs baseline: 1.9833x; 1.0731x over previous
import jax
import jax.numpy as jnp
from jax import lax
from jax.experimental import pallas as pl
from jax.experimental.pallas import tpu as pltpu
from jax.experimental.pallas import tpu_sc as plsc

_NW = 32
_L = 16


def kernel(weeks, seasonal_bias):
    n = weeks.shape[0]
    v, d0, d1 = seasonal_bias.shape
    planes = d0 * d1
    per_w = n // _NW
    idx = weeks.astype(jnp.int32)
    table_t = seasonal_bias.reshape(v, planes).T.reshape(-1)

    mesh = plsc.VectorSubcoreMesh(core_axis_name="c", subcore_axis_name="s")

    @pl.kernel(
        out_type=jax.ShapeDtypeStruct((d0, d1, n), seasonal_bias.dtype),
        mesh=mesh,
        scratch_types=[
            pltpu.VMEM((planes * v,), jnp.float32),
            pltpu.VMEM((per_w,), jnp.int32),
            pltpu.VMEM((d0, d1, per_w), jnp.float32),
            pltpu.SemaphoreType.DMA((d0,)),
            pltpu.SemaphoreType.DMA,
        ],
        compiler_params=pltpu.CompilerParams(needs_layout_passes=False),
    )
    def gather_kernel(
        table_hbm, idx_hbm, out_hbm, table_v, idx_v, pane_v, osem, tsem
    ):
        wid = lax.axis_index("s") * 2 + lax.axis_index("c")
        base = wid * per_w
        tcopy = pltpu.make_async_copy(table_hbm, table_v, tsem)
        tcopy.start()
        pltpu.sync_copy(idx_hbm.at[pl.ds(base, per_w)], idx_v)
        tcopy.wait()

        @plsc.parallel_loop(0, per_w // _L, unroll=2)
        def _(c):
            w = idx_v[pl.ds(c * _L, _L)]
            gidx = w
            for p in range(planes):
                vals = plsc.load_gather(table_v, [gidx])
                pane_v[p // d1, p % d1, pl.ds(c * _L, _L)] = vals
                gidx = gidx + v

        outs = []
        for d in range(d0):
            o = pltpu.make_async_copy(
                pane_v.at[d],
                out_hbm.at[d, :, pl.ds(base, per_w)],
                osem.at[d],
            )
            o.start()
            outs.append(o)
        for o in outs:
            o.wait()

    out_t = gather_kernel(table_t, idx)
    return jnp.transpose(out_t, (2, 0, 1))


# speedup vs baseline: 2.7399x; 1.3815x over previous
import jax
import jax.numpy as jnp
from jax import lax
from jax.experimental import pallas as pl
from jax.experimental.pallas import tpu as pltpu
from jax.experimental.pallas import tpu_sc as plsc

_NW = 32
_L = 16


def kernel(weeks, seasonal_bias):
    n = weeks.shape[0]
    v, d0, d1 = seasonal_bias.shape
    planes = d0 * d1
    per_w = n // _NW
    idx = weeks.astype(jnp.int32)
    table_t = seasonal_bias.reshape(v, planes).T.reshape(-1)

    mesh = plsc.VectorSubcoreMesh(core_axis_name="c", subcore_axis_name="s")

    @pl.kernel(
        out_type=jax.ShapeDtypeStruct((d0, d1, n), seasonal_bias.dtype),
        mesh=mesh,
        scratch_types=[
            pltpu.VMEM((planes * v,), jnp.float32),
            pltpu.VMEM((per_w,), jnp.int32),
            pltpu.VMEM((d0, d1, per_w), jnp.float32),
            pltpu.SemaphoreType.DMA((d0,)),
            pltpu.SemaphoreType.DMA,
        ],
        compiler_params=pltpu.CompilerParams(needs_layout_passes=False),
    )
    def gather_kernel(
        table_hbm, idx_hbm, out_hbm, table_v, idx_v, pane_v, osem, tsem
    ):
        wid = lax.axis_index("s") * 2 + lax.axis_index("c")
        base = wid * per_w
        tcopy = pltpu.make_async_copy(table_hbm, table_v, tsem)
        tcopy.start()
        pltpu.sync_copy(idx_hbm.at[pl.ds(base, per_w)], idx_v)
        tcopy.wait()

        outs = []
        for d in range(d0):

            @plsc.parallel_loop(0, per_w // _L, unroll=2)
            def _(c, d=d):
                w = idx_v[pl.ds(c * _L, _L)]
                gidx = w + (d * d1) * v
                for h in range(d1):
                    vals = plsc.load_gather(table_v, [gidx])
                    pane_v[d, h, pl.ds(c * _L, _L)] = vals
                    gidx = gidx + v

            o = pltpu.make_async_copy(
                pane_v.at[d],
                out_hbm.at[d, :, pl.ds(base, per_w)],
                osem.at[d],
            )
            o.start()
            outs.append(o)
        for o in outs:
            o.wait()

    out_t = gather_kernel(table_t, idx)
    return jnp.transpose(out_t, (2, 0, 1))
